# baseline (speedup 1.0000x reference)
import functools

import jax
import jax.numpy as jnp
from jax import lax
from jax.experimental import pallas as pl
from jax.experimental.pallas import tpu as pltpu
from jax.experimental.pallas import tpu_sc as plsc

BATCH = 4
NROWS = 8192
DIM = 1024
NC, NS, L = 2, 16, 16
NW = NC * NS
ROWS_PER_W = NROWS // NW
CH = 32
CHW = CH * DIM
NCHUNK = ROWS_PER_W // CH
UNROLL = 8

_mesh = plsc.VectorSubcoreMesh(
    core_axis_name="c", subcore_axis_name="s", num_cores=NC, num_subcores=NS
)


@functools.partial(
    pl.kernel,
    out_type=jax.ShapeDtypeStruct((BATCH * NROWS * DIM,), jnp.float32),
    mesh=_mesh,
    scratch_types=[
        pltpu.VMEM((CHW,), jnp.float32),
        pltpu.VMEM((CHW,), jnp.float32),
    ],
)
def _sc_add(x_hbm, pos_hbm, out_hbm, pos_v, work_v):
    wid = lax.axis_index("s") * NC + lax.axis_index("c")
    base = wid * (ROWS_PER_W * DIM)

    def chunk_body(c, carry):
        pos_off = base + c * CHW
        pltpu.sync_copy(pos_hbm.at[pl.ds(pos_off, CHW)], pos_v)

        def batch_body(b, carry):
            xoff = b * (NROWS * DIM) + pos_off
            pltpu.sync_copy(x_hbm.at[pl.ds(xoff, CHW)], work_v)

            def add_body(i, carry):
                o = i * (L * UNROLL)
                for k in range(UNROLL):
                    s = pl.ds(o + k * L, L)
                    plsc.addupdate(work_v.at[s], pos_v[s])
                return carry

            lax.fori_loop(0, CHW // (L * UNROLL), add_body, 0)
            pltpu.sync_copy(work_v, out_hbm.at[pl.ds(xoff, CHW)])
            return carry

        lax.fori_loop(0, BATCH, batch_body, 0)
        return carry

    lax.fori_loop(0, NCHUNK, chunk_body, 0)


def kernel(x, pos_emb):
    flat = _sc_add(x.reshape(-1), pos_emb.reshape(-1))
    return flat.reshape(x.shape)


# speedup vs baseline: 1.1374x; 1.1374x over previous
import functools

import jax
import jax.numpy as jnp
from jax import lax
from jax.experimental import pallas as pl
from jax.experimental.pallas import tpu as pltpu
from jax.experimental.pallas import tpu_sc as plsc

BATCH = 4
NROWS = 8192
DIM = 1024
NC, NS, L = 2, 16, 16
NW = NC * NS
ROWS_PER_W = NROWS // NW
CH = 16
CHW = CH * DIM
NCHUNK = ROWS_PER_W // CH
UNROLL = 8

_mesh = plsc.VectorSubcoreMesh(
    core_axis_name="c", subcore_axis_name="s", num_cores=NC, num_subcores=NS
)


@functools.partial(
    pl.kernel,
    out_type=jax.ShapeDtypeStruct((BATCH * NROWS * DIM,), jnp.float32),
    mesh=_mesh,
    scratch_types=[
        [pltpu.VMEM((CHW,), jnp.float32) for _ in range(BATCH)],
        [pltpu.VMEM((CHW,), jnp.float32) for _ in range(2)],
        [pltpu.SemaphoreType.DMA for _ in range(BATCH)],
        [pltpu.SemaphoreType.DMA for _ in range(BATCH)],
        [pltpu.SemaphoreType.DMA for _ in range(2)],
    ],
)
def _sc_add(x_hbm, pos_hbm, out_hbm, bufs, pos_bufs, in_sems, out_sems, pos_sems):
    wid = lax.axis_index("s") * NC + lax.axis_index("c")
    base = wid * (ROWS_PER_W * DIM)

    def pos_slice(c):
        return pl.ds(base + c * CHW, CHW)

    def x_slice(c, b):
        return pl.ds(b * (NROWS * DIM) + base + c * CHW, CHW)

    pos_handles = [
        pltpu.async_copy(pos_hbm.at[pos_slice(0)], pos_bufs[0], pos_sems[0]),
        pltpu.async_copy(pos_hbm.at[pos_slice(1)], pos_bufs[1], pos_sems[1]),
    ]
    pending_out = [None] * BATCH

    for c in range(NCHUNK):
        pc = pos_bufs[c % 2]
        in_handles = []
        for b in range(BATCH):
            if pending_out[b] is not None:
                pending_out[b].wait()
            in_handles.append(
                pltpu.async_copy(x_hbm.at[x_slice(c, b)], bufs[b], in_sems[b])
            )
        if 1 <= c < NCHUNK - 1:
            nc = c + 1
            pos_handles[nc % 2] = pltpu.async_copy(
                pos_hbm.at[pos_slice(nc)], pos_bufs[nc % 2], pos_sems[nc % 2]
            )
        pos_handles[c % 2].wait()
        for b in range(BATCH):
            in_handles[b].wait()
            buf = bufs[b]

            def add_body(i, buf=buf):
                for k in range(UNROLL):
                    s = pl.ds(i + k * L, L)
                    plsc.addupdate(buf.at[s], pc[s])

            plsc.parallel_loop(0, CHW, L * UNROLL)(add_body)

            pending_out[b] = pltpu.async_copy(
                buf, out_hbm.at[x_slice(c, b)], out_sems[b]
            )

    for b in range(BATCH):
        pending_out[b].wait()


def kernel(x, pos_emb):
    flat = _sc_add(x.reshape(-1), pos_emb.reshape(-1))
    return flat.reshape(x.shape)
